# Initial kernel scaffold; baseline (speedup 1.0000x reference)
import jax
import jax.numpy as jnp
from jax.experimental import pallas as pl


def kernel(historical_review, review_positive, review_negative, user, item, label, user_histor_idx, item_histor_idx, emb_table, M_w, W_w, W_b, T_w):
    raise NotImplementedError("write your pallas kernel here")


# speedup vs baseline: 1.0902x; 1.0902x over previous
import functools

import jax
import jax.numpy as jnp
from jax import lax
from jax.experimental import pallas as pl
from jax.experimental.pallas import tpu as pltpu
from jax.experimental.pallas import tpu_sc as plsc

N = 4096
L = 50
D = 64
A = 32
NEG = 5
P = 1024
H = 8
AVG_RATING = 3.5
EPS = 1e-12
BN = 256
NC = 2
NS = 16
NW = NC * NS
GCH = 128


@functools.lru_cache(maxsize=None)
def _make_sc_gather(n_idx: int):
    b_per_w = n_idx // NW
    nchunk = b_per_w // GCH
    mesh = plsc.VectorSubcoreMesh(core_axis_name="c", subcore_axis_name="s")

    @functools.partial(
        pl.kernel,
        mesh=mesh,
        out_type=jax.ShapeDtypeStruct((n_idx, D), jnp.float32),
        scratch_types=[
            pltpu.VMEM((GCH,), jnp.int32),
            pltpu.VMEM((GCH, D), jnp.float32),
            pltpu.SemaphoreType.DMA,
        ],
        compiler_params=pltpu.CompilerParams(use_tc_tiling_on_sc=False),
    )
    def gather_kernel(table_hbm, idx_hbm, out_hbm, idx_v, rows_v, sem):
        wid = lax.axis_index("s") * NC + lax.axis_index("c")
        base = wid * b_per_w

        def body(c, carry):
            off = base + c * GCH
            pltpu.sync_copy(idx_hbm.at[pl.ds(off, GCH)], idx_v)
            pltpu.async_copy(table_hbm.at[idx_v], rows_v, sem).wait()
            pltpu.sync_copy(rows_v, out_hbm.at[pl.ds(off, GCH)])
            return carry

        lax.fori_loop(0, nchunk, body, 0)

    return gather_kernel


def _main_body(ew_ref, rp_ref, zn_ref, mw_ref, ww_ref, wb_ref, tw_ref,
               t1_ref, s1_ref, t2_ref, s_ref, rs_ref, j_ref):
    ew = ew_ref[...]
    q = jnp.dot(rp_ref[...], mw_ref[...])
    qt = jnp.dot(q, t1_ref[...])
    dx = jnp.dot(ew * qt, s1_ref[...])
    m = jnp.max(dx, axis=1, keepdims=True)
    e = jnp.exp(dx - m)
    ax = e / jnp.sum(e, axis=1, keepdims=True)
    axt = jnp.dot(ax, t2_ref[...])
    zs = jnp.dot(ew * axt, s_ref[...])
    logits = lax.dot_general(zs, ww_ref[...],
                             (((1,), (1,)), ((), ()))) + wb_ref[...]
    pm = jnp.max(logits, axis=1, keepdims=True)
    pe = jnp.exp(logits - pm)
    pt = pe / jnp.sum(pe, axis=1, keepdims=True)
    rs = lax.dot_general(pt, tw_ref[...], (((1,), (1,)), ((), ())))
    rs_ref[...] = rs
    rnorm = jnp.maximum(jnp.sqrt(jnp.sum(rs * rs, axis=1, keepdims=True)), EPS)
    znorm = jnp.maximum(jnp.sqrt(jnp.sum(zs * zs, axis=1, keepdims=True)), EPS)
    c1 = jnp.sum(rs * zs, axis=1, keepdims=True) / (rnorm * znorm)
    rhat = rs / rnorm
    acc = jnp.zeros((BN, 1), jnp.float32)
    for g in range(NEG):
        seg = zn_ref[:, g * D:(g + 1) * D]
        snorm = jnp.maximum(
            jnp.sqrt(jnp.sum(seg * seg, axis=1, keepdims=True)), EPS)
        c2 = jnp.sum(seg * rhat, axis=1, keepdims=True) / snorm
        acc = acc + jnp.maximum(c2 - c1, 0.0)

    @pl.when(pl.program_id(0) == 0)
    def _init():
        j_ref[...] = jnp.zeros_like(j_ref)

    j_ref[...] += jnp.sum(acc, keepdims=True)


def _final_body(gu_ref, gi_ref, lab_ref, tw_ref, jin_ref,
                obj_ref, rl_ref, ab_ref):
    i = pl.program_id(0)
    nb = pl.num_programs(0)
    ue = jnp.zeros((P // 4, D), jnp.float32)
    ie = jnp.zeros((P // 4, D), jnp.float32)
    for h in range(H):
        ue = ue + gu_ref[:, h * D:(h + 1) * D]
        ie = ie + gi_ref[:, h * D:(h + 1) * D]
    ue = ue * (1.0 / H)
    ie = ie * (1.0 / H)
    pred = jnp.sum(ue * ie, axis=1, keepdims=True) + AVG_RATING
    diff = pred - lab_ref[...]

    @pl.when(i == 0)
    def _init():
        rl_ref[...] = jnp.zeros_like(rl_ref)

    rl_ref[...] += jnp.sum(diff * diff, keepdims=True)

    @pl.when(i == nb - 1)
    def _fin():
        tw = tw_ref[...]
        ss = jnp.sum(tw * tw, axis=0, keepdims=True)
        tn = tw / jnp.maximum(jnp.sqrt(ss), EPS)
        u = lax.dot_general(tn, tn, (((0,), (0,)), ((), ())))
        row = lax.broadcasted_iota(jnp.int32, (A, A), 0)
        col = lax.broadcasted_iota(jnp.int32, (A, A), 1)
        eye = jnp.where(row == col, 1.0, 0.0)
        ul = jnp.sum((u - eye) ** 2, keepdims=True) * (1.0 / (A * A))
        jl = jin_ref[...] * (1.0 / (N * NEG))
        ab = ul + jl
        rl = rl_ref[...] * (1.0 / P)
        rl_ref[...] = rl
        ab_ref[...] = ab
        obj_ref[...] = rl + ab


def _main_call(ew2, rp, zn2, mw, ww, wb2, tw, t1, s1, t2, s):
    return pl.pallas_call(
        _main_body,
        grid=(N // BN,),
        in_specs=[
            pl.BlockSpec((BN, L * D), lambda i: (i, 0)),
            pl.BlockSpec((BN, D), lambda i: (i, 0)),
            pl.BlockSpec((BN, NEG * D), lambda i: (i, 0)),
            pl.BlockSpec((D, D), lambda i: (0, 0)),
            pl.BlockSpec((A, D), lambda i: (0, 0)),
            pl.BlockSpec((1, A), lambda i: (0, 0)),
            pl.BlockSpec((D, A), lambda i: (0, 0)),
            pl.BlockSpec((D, L * D), lambda i: (0, 0)),
            pl.BlockSpec((L * D, L), lambda i: (0, 0)),
            pl.BlockSpec((L, L * D), lambda i: (0, 0)),
            pl.BlockSpec((L * D, D), lambda i: (0, 0)),
        ],
        out_specs=[
            pl.BlockSpec((BN, D), lambda i: (i, 0)),
            pl.BlockSpec((1, 1), lambda i: (0, 0)),
        ],
        out_shape=[
            jax.ShapeDtypeStruct((N, D), jnp.float32),
            jax.ShapeDtypeStruct((1, 1), jnp.float32),
        ],
        compiler_params=pltpu.CompilerParams(
            dimension_semantics=("arbitrary",)),
    )(ew2, rp, zn2, mw, ww, wb2, tw, t1, s1, t2, s)


def _final_call(gv, lab, tw, jsum):
    nb = 4
    bp = P // nb
    return pl.pallas_call(
        _final_body,
        grid=(nb,),
        in_specs=[
            pl.BlockSpec((bp, H * D), lambda i: (i, 0)),
            pl.BlockSpec((bp, H * D), lambda i: (i + nb, 0)),
            pl.BlockSpec((bp, 1), lambda i: (i, 0)),
            pl.BlockSpec((D, A), lambda i: (0, 0)),
            pl.BlockSpec((1, 1), lambda i: (0, 0)),
        ],
        out_specs=[
            pl.BlockSpec((1, 1), lambda i: (0, 0)),
            pl.BlockSpec((1, 1), lambda i: (0, 0)),
            pl.BlockSpec((1, 1), lambda i: (0, 0)),
        ],
        out_shape=[
            jax.ShapeDtypeStruct((1, 1), jnp.float32),
            jax.ShapeDtypeStruct((1, 1), jnp.float32),
            jax.ShapeDtypeStruct((1, 1), jnp.float32),
        ],
        compiler_params=pltpu.CompilerParams(
            dimension_semantics=("arbitrary",)),
    )(gv, gv, lab, tw, jsum)


def _constants():
    jidx = jnp.arange(L * D)
    t1 = (jidx[None, :] % D == jnp.arange(D)[:, None]).astype(jnp.float32)
    s1 = (jidx[:, None] // D == jnp.arange(L)[None, :]).astype(jnp.float32)
    t2 = (jidx[None, :] % L == jnp.arange(L)[:, None]).astype(jnp.float32)
    s = (jidx[:, None] // L == jnp.arange(D)[None, :]).astype(jnp.float32)
    return t1, s1, t2, s


def kernel(historical_review, review_positive, review_negative, user, item,
           label, user_histor_idx, item_histor_idx,
           emb_table, M_w, W_w, W_b, T_w):
    hr = historical_review.reshape(-1).astype(jnp.int32)
    ew_flat = _make_sc_gather(N * L)(emb_table, hr)
    ew2 = ew_flat.reshape(N, L * D)
    zn2 = review_negative.reshape(N, NEG * D)
    wb2 = W_b.reshape(1, A)
    t1, s1, t2, s = _constants()
    rs, jsum = _main_call(ew2, review_positive, zn2, M_w, W_w, wb2, T_w,
                          t1, s1, t2, s)
    hidx = jnp.concatenate([user_histor_idx.reshape(-1),
                            item_histor_idx.reshape(-1)]).astype(jnp.int32)
    gh = _make_sc_gather(2 * P * H)(rs, hidx)
    gv = gh.reshape(2 * P, H * D)
    lab = label.reshape(P, 1)
    obj, rl, ab = _final_call(gv, lab, T_w, jsum)
    return (obj[0, 0], rl[0, 0], ab[0, 0])


# speedup vs baseline: 1.6658x; 1.5279x over previous
import functools

import jax
import jax.numpy as jnp
from jax import lax
from jax.experimental import pallas as pl
from jax.experimental.pallas import tpu as pltpu
from jax.experimental.pallas import tpu_sc as plsc

N = 4096
L = 50
D = 64
A = 32
NEG = 5
P = 1024
H = 8
AVG_RATING = 3.5
EPS = 1e-12
BN = 256
NC = 2
NS = 16
NW = NC * NS
GCH = 128


@functools.lru_cache(maxsize=None)
def _make_sc_rowdma_gather(n_idx: int, n_tiles: int):
    b_per_w = n_idx // NW
    nchunk = b_per_w // GCH
    mesh = plsc.VectorSubcoreMesh(core_axis_name="c", subcore_axis_name="s")

    @functools.partial(
        pl.kernel,
        mesh=mesh,
        out_type=jax.ShapeDtypeStruct((n_idx, D), jnp.float32),
        scratch_types=[
            pltpu.VMEM((GCH,), jnp.int32),
            pltpu.VMEM((GCH, D), jnp.float32),
            pltpu.SemaphoreType.DMA,
        ],
    )
    def gather_kernel(table_hbm, idx_hbm, out_hbm, idx_v, rows_v, sem):
        wid = lax.axis_index("s") * NC + lax.axis_index("c")
        base = wid * b_per_w

        def chunk_body(c, carry):
            off = base + c * GCH
            pltpu.sync_copy(idx_hbm.at[pl.ds(off, GCH)], idx_v)

            def fire16(c2, carry2):
                vec = idx_v[pl.ds(c2 * 16, 16)]
                for j in range(16):
                    i = vec[j]
                    pltpu.make_async_copy(
                        table_hbm.at[i // 8, i % 8],
                        rows_v.at[c2 * 16 + j], sem).start()
                return carry2

            lax.fori_loop(0, GCH // 16, fire16, 0)
            pltpu.make_async_copy(
                out_hbm.at[pl.ds(0, GCH)], rows_v, sem).wait()
            pltpu.sync_copy(rows_v, out_hbm.at[pl.ds(off, GCH)])
            return carry

        lax.fori_loop(0, nchunk, chunk_body, 0)

    return gather_kernel


@functools.lru_cache(maxsize=None)
def _make_sc_gather(n_idx: int):
    b_per_w = n_idx // NW
    nchunk = b_per_w // GCH
    mesh = plsc.VectorSubcoreMesh(core_axis_name="c", subcore_axis_name="s")

    @functools.partial(
        pl.kernel,
        mesh=mesh,
        out_type=jax.ShapeDtypeStruct((n_idx, 2 * D), jnp.float32),
        scratch_types=[
            pltpu.VMEM((GCH,), jnp.int32),
            pltpu.VMEM((GCH, 2 * D), jnp.float32),
            pltpu.SemaphoreType.DMA,
        ],
    )
    def gather_kernel(table_hbm, idx_hbm, out_hbm, idx_v, rows_v, sem):
        wid = lax.axis_index("s") * NC + lax.axis_index("c")
        base = wid * b_per_w

        def body(c, carry):
            off = base + c * GCH
            pltpu.sync_copy(idx_hbm.at[pl.ds(off, GCH)], idx_v)
            pltpu.async_copy(table_hbm.at[idx_v], rows_v, sem).wait()
            pltpu.sync_copy(rows_v, out_hbm.at[pl.ds(off, GCH)])
            return carry

        lax.fori_loop(0, nchunk, body, 0)

    return gather_kernel


def _main_body(ew_ref, rp_ref, zn_ref, mw_ref, ww_ref, wb_ref, tw_ref,
               t1_ref, s1_ref, t2_ref, s_ref, rs_ref, j_ref):
    ew = ew_ref[...]
    q = jnp.dot(rp_ref[...], mw_ref[...])
    qt = jnp.dot(q, t1_ref[...])
    dx = jnp.dot(ew * qt, s1_ref[...])
    m = jnp.max(dx, axis=1, keepdims=True)
    e = jnp.exp(dx - m)
    ax = e / jnp.sum(e, axis=1, keepdims=True)
    axt = jnp.dot(ax, t2_ref[...])
    zs = jnp.dot(ew * axt, s_ref[...])
    logits = lax.dot_general(zs, ww_ref[...],
                             (((1,), (1,)), ((), ()))) + wb_ref[...]
    pm = jnp.max(logits, axis=1, keepdims=True)
    pe = jnp.exp(logits - pm)
    pt = pe / jnp.sum(pe, axis=1, keepdims=True)
    rs = lax.dot_general(pt, tw_ref[...], (((1,), (1,)), ((), ())))
    rs_ref[:, :D] = rs
    rs_ref[:, D:] = jnp.zeros((BN, D), jnp.float32)
    rnorm = jnp.maximum(jnp.sqrt(jnp.sum(rs * rs, axis=1, keepdims=True)), EPS)
    znorm = jnp.maximum(jnp.sqrt(jnp.sum(zs * zs, axis=1, keepdims=True)), EPS)
    c1 = jnp.sum(rs * zs, axis=1, keepdims=True) / (rnorm * znorm)
    rhat = rs / rnorm
    acc = jnp.zeros((BN, 1), jnp.float32)
    for g in range(NEG):
        seg = zn_ref[:, g * D:(g + 1) * D]
        snorm = jnp.maximum(
            jnp.sqrt(jnp.sum(seg * seg, axis=1, keepdims=True)), EPS)
        c2 = jnp.sum(seg * rhat, axis=1, keepdims=True) / snorm
        acc = acc + jnp.maximum(c2 - c1, 0.0)

    @pl.when(pl.program_id(0) == 0)
    def _init():
        j_ref[...] = jnp.zeros_like(j_ref)

    j_ref[...] += jnp.sum(acc, keepdims=True)


def _final_body(gu_ref, gi_ref, lab_ref, tw_ref, jin_ref,
                obj_ref, rl_ref, ab_ref):
    i = pl.program_id(0)
    nb = pl.num_programs(0)
    bp = P // 4
    ue = jnp.sum(gu_ref[...].reshape(bp, H, 2 * D), axis=1) * (1.0 / H)
    ie = jnp.sum(gi_ref[...].reshape(bp, H, 2 * D), axis=1) * (1.0 / H)
    pred = jnp.sum(ue * ie, axis=1, keepdims=True) + AVG_RATING
    diff = pred - lab_ref[...]

    @pl.when(i == 0)
    def _init():
        rl_ref[...] = jnp.zeros_like(rl_ref)

    rl_ref[...] += jnp.sum(diff * diff, keepdims=True)

    @pl.when(i == nb - 1)
    def _fin():
        tw = tw_ref[...]
        ss = jnp.sum(tw * tw, axis=0, keepdims=True)
        tn = tw / jnp.maximum(jnp.sqrt(ss), EPS)
        u = lax.dot_general(tn, tn, (((0,), (0,)), ((), ())))
        row = lax.broadcasted_iota(jnp.int32, (A, A), 0)
        col = lax.broadcasted_iota(jnp.int32, (A, A), 1)
        eye = jnp.where(row == col, 1.0, 0.0)
        ul = jnp.sum((u - eye) ** 2, keepdims=True) * (1.0 / (A * A))
        jl = jin_ref[...] * (1.0 / (N * NEG))
        ab = ul + jl
        rl = rl_ref[...] * (1.0 / P)
        rl_ref[...] = rl
        ab_ref[...] = ab
        obj_ref[...] = rl + ab


def _main_call(ew2, rp, zn2, mw, ww, wb2, tw, t1, s1, t2, s):
    return pl.pallas_call(
        _main_body,
        grid=(N // BN,),
        in_specs=[
            pl.BlockSpec((BN, L * D), lambda i: (i, 0)),
            pl.BlockSpec((BN, D), lambda i: (i, 0)),
            pl.BlockSpec((BN, NEG * D), lambda i: (i, 0)),
            pl.BlockSpec((D, D), lambda i: (0, 0)),
            pl.BlockSpec((A, D), lambda i: (0, 0)),
            pl.BlockSpec((1, A), lambda i: (0, 0)),
            pl.BlockSpec((D, A), lambda i: (0, 0)),
            pl.BlockSpec((D, L * D), lambda i: (0, 0)),
            pl.BlockSpec((L * D, L), lambda i: (0, 0)),
            pl.BlockSpec((L, L * D), lambda i: (0, 0)),
            pl.BlockSpec((L * D, D), lambda i: (0, 0)),
        ],
        out_specs=[
            pl.BlockSpec((BN, 2 * D), lambda i: (i, 0)),
            pl.BlockSpec((1, 1), lambda i: (0, 0)),
        ],
        out_shape=[
            jax.ShapeDtypeStruct((N, 2 * D), jnp.float32),
            jax.ShapeDtypeStruct((1, 1), jnp.float32),
        ],
        compiler_params=pltpu.CompilerParams(
            dimension_semantics=("arbitrary",)),
    )(ew2, rp, zn2, mw, ww, wb2, tw, t1, s1, t2, s)


def _final_call(gh, lab, tw, jsum):
    nb = 4
    bp = P // nb
    return pl.pallas_call(
        _final_body,
        grid=(nb,),
        in_specs=[
            pl.BlockSpec((bp * H, 2 * D), lambda i: (i, 0)),
            pl.BlockSpec((bp * H, 2 * D), lambda i: (i + nb, 0)),
            pl.BlockSpec((bp, 1), lambda i: (i, 0)),
            pl.BlockSpec((D, A), lambda i: (0, 0)),
            pl.BlockSpec((1, 1), lambda i: (0, 0)),
        ],
        out_specs=[
            pl.BlockSpec((1, 1), lambda i: (0, 0)),
            pl.BlockSpec((1, 1), lambda i: (0, 0)),
            pl.BlockSpec((1, 1), lambda i: (0, 0)),
        ],
        out_shape=[
            jax.ShapeDtypeStruct((1, 1), jnp.float32),
            jax.ShapeDtypeStruct((1, 1), jnp.float32),
            jax.ShapeDtypeStruct((1, 1), jnp.float32),
        ],
        compiler_params=pltpu.CompilerParams(
            dimension_semantics=("arbitrary",)),
    )(gh, gh, lab, tw, jsum)


def _constants():
    jidx = jnp.arange(L * D)
    t1 = (jidx[None, :] % D == jnp.arange(D)[:, None]).astype(jnp.float32)
    s1 = (jidx[:, None] // D == jnp.arange(L)[None, :]).astype(jnp.float32)
    t2 = (jidx[None, :] % L == jnp.arange(L)[:, None]).astype(jnp.float32)
    s = (jidx[:, None] // L == jnp.arange(D)[None, :]).astype(jnp.float32)
    return t1, s1, t2, s


def kernel(historical_review, review_positive, review_negative, user, item,
           label, user_histor_idx, item_histor_idx,
           emb_table, M_w, W_w, W_b, T_w):
    hr = historical_review.reshape(-1).astype(jnp.int32)
    n_tiles = emb_table.shape[0] // 8
    emb3 = emb_table.reshape(n_tiles, 8, D)
    ew_flat = _make_sc_rowdma_gather(N * L, n_tiles)(emb3, hr)
    ew2 = ew_flat.reshape(N, L * D)
    zn2 = review_negative.reshape(N, NEG * D)
    wb2 = W_b.reshape(1, A)
    t1, s1, t2, s = _constants()
    rs, jsum = _main_call(ew2, review_positive, zn2, M_w, W_w, wb2, T_w,
                          t1, s1, t2, s)
    hidx = jnp.concatenate([user_histor_idx.reshape(-1),
                            item_histor_idx.reshape(-1)]).astype(jnp.int32)
    gh = _make_sc_gather(2 * P * H)(rs, hidx)
    lab = label.reshape(P, 1)
    obj, rl, ab = _final_call(gh, lab, T_w, jsum)
    return (obj[0, 0], rl[0, 0], ab[0, 0])


# speedup vs baseline: 2.1706x; 1.3030x over previous
import functools

import jax
import jax.numpy as jnp
from jax import lax
from jax.experimental import pallas as pl
from jax.experimental.pallas import tpu as pltpu
from jax.experimental.pallas import tpu_sc as plsc

N = 4096
L = 50
D = 64
A = 32
NEG = 5
P = 1024
H = 8
AVG_RATING = 3.5
EPS = 1e-12
BN = 256
NC = 2
NS = 16
NW = NC * NS
GCH = 128


GRP = 8 * L


@functools.lru_cache(maxsize=None)
def _make_sc_rowdma_gather(n_rev: int, n_tiles: int):
    rev_per_w = n_rev // NW
    nchunk = rev_per_w // 8
    mesh = plsc.VectorSubcoreMesh(core_axis_name="c", subcore_axis_name="s")

    @functools.partial(
        pl.kernel,
        mesh=mesh,
        out_type=jax.ShapeDtypeStruct((n_rev // 8, 8, L * D), jnp.float32),
        scratch_types=[
            pltpu.VMEM((GRP,), jnp.int32),
            pltpu.VMEM((8, L * D), jnp.float32),
            pltpu.SemaphoreType.DMA,
        ],
    )
    def gather_kernel(table_hbm, idx_hbm, out_hbm, idx_v, rows_v, sem):
        wid = lax.axis_index("s") * NC + lax.axis_index("c")
        base = wid * rev_per_w * L
        gbase = wid * nchunk

        def chunk_body(c, carry):
            off = base + c * GRP
            pltpu.sync_copy(idx_hbm.at[pl.ds(off, GRP)], idx_v)

            def fire16(c2, qr):
                q, r = qr
                vec = idx_v[pl.ds(c2 * 16, 16)]
                for j in range(16):
                    i = vec[j]
                    pltpu.make_async_copy(
                        table_hbm.at[i // 8, i % 8],
                        rows_v.at[q, pl.ds(r * D, D)], sem).start()
                    wrap = r == (L - 1)
                    q = jnp.where(wrap, q + 1, q)
                    r = jnp.where(wrap, 0, r + 1)
                return (q, r)

            lax.fori_loop(0, GRP // 16, fire16,
                          (jnp.int32(0), jnp.int32(0)))
            pltpu.make_async_copy(
                out_hbm.at[gbase + c], rows_v, sem).wait()
            for s in range(8):
                pltpu.sync_copy(rows_v.at[s], out_hbm.at[gbase + c, s])
            return carry

        lax.fori_loop(0, nchunk, chunk_body, 0)

    return gather_kernel


@functools.lru_cache(maxsize=None)
def _make_sc_gather(n_idx: int):
    b_per_w = n_idx // NW
    nchunk = b_per_w // GCH
    mesh = plsc.VectorSubcoreMesh(core_axis_name="c", subcore_axis_name="s")

    @functools.partial(
        pl.kernel,
        mesh=mesh,
        out_type=jax.ShapeDtypeStruct((n_idx, 2 * D), jnp.float32),
        scratch_types=[
            pltpu.VMEM((GCH,), jnp.int32),
            pltpu.VMEM((GCH, 2 * D), jnp.float32),
            pltpu.SemaphoreType.DMA,
        ],
    )
    def gather_kernel(table_hbm, idx_hbm, out_hbm, idx_v, rows_v, sem):
        wid = lax.axis_index("s") * NC + lax.axis_index("c")
        base = wid * b_per_w

        def body(c, carry):
            off = base + c * GCH
            pltpu.sync_copy(idx_hbm.at[pl.ds(off, GCH)], idx_v)
            pltpu.async_copy(table_hbm.at[idx_v], rows_v, sem).wait()
            pltpu.sync_copy(rows_v, out_hbm.at[pl.ds(off, GCH)])
            return carry

        lax.fori_loop(0, nchunk, body, 0)

    return gather_kernel


def _main_body(ew_ref, rp_ref, zn_ref, mw_ref, ww_ref, wb_ref, tw_ref,
               t1_ref, s1_ref, t2_ref, s_ref, rs_ref, j_ref):
    ew = ew_ref[...]
    q = jnp.dot(rp_ref[...], mw_ref[...])
    qt = jnp.dot(q, t1_ref[...])
    dx = jnp.dot(ew * qt, s1_ref[...])
    m = jnp.max(dx, axis=1, keepdims=True)
    e = jnp.exp(dx - m)
    ax = e / jnp.sum(e, axis=1, keepdims=True)
    axt = jnp.dot(ax, t2_ref[...])
    zs = jnp.dot(ew * axt, s_ref[...])
    logits = lax.dot_general(zs, ww_ref[...],
                             (((1,), (1,)), ((), ()))) + wb_ref[...]
    pm = jnp.max(logits, axis=1, keepdims=True)
    pe = jnp.exp(logits - pm)
    pt = pe / jnp.sum(pe, axis=1, keepdims=True)
    rs = lax.dot_general(pt, tw_ref[...], (((1,), (1,)), ((), ())))
    rs_ref[:, :D] = rs
    rs_ref[:, D:] = jnp.zeros((BN, D), jnp.float32)
    rnorm = jnp.maximum(jnp.sqrt(jnp.sum(rs * rs, axis=1, keepdims=True)), EPS)
    znorm = jnp.maximum(jnp.sqrt(jnp.sum(zs * zs, axis=1, keepdims=True)), EPS)
    c1 = jnp.sum(rs * zs, axis=1, keepdims=True) / (rnorm * znorm)
    rhat = rs / rnorm
    acc = jnp.zeros((BN, 1), jnp.float32)
    for g in range(NEG):
        seg = zn_ref[:, g * D:(g + 1) * D]
        snorm = jnp.maximum(
            jnp.sqrt(jnp.sum(seg * seg, axis=1, keepdims=True)), EPS)
        c2 = jnp.sum(seg * rhat, axis=1, keepdims=True) / snorm
        acc = acc + jnp.maximum(c2 - c1, 0.0)

    @pl.when(pl.program_id(0) == 0)
    def _init():
        j_ref[...] = jnp.zeros_like(j_ref)

    j_ref[...] += jnp.sum(acc, keepdims=True)


def _final_body(gu_ref, gi_ref, lab_ref, tw_ref, jin_ref,
                obj_ref, rl_ref, ab_ref):
    i = pl.program_id(0)
    nb = pl.num_programs(0)
    bp = P // 4
    ue = jnp.sum(gu_ref[...].reshape(bp, H, 2 * D), axis=1) * (1.0 / H)
    ie = jnp.sum(gi_ref[...].reshape(bp, H, 2 * D), axis=1) * (1.0 / H)
    pred = jnp.sum(ue * ie, axis=1, keepdims=True) + AVG_RATING
    diff = pred - lab_ref[...]

    @pl.when(i == 0)
    def _init():
        rl_ref[...] = jnp.zeros_like(rl_ref)

    rl_ref[...] += jnp.sum(diff * diff, keepdims=True)

    @pl.when(i == nb - 1)
    def _fin():
        tw = tw_ref[...]
        ss = jnp.sum(tw * tw, axis=0, keepdims=True)
        tn = tw / jnp.maximum(jnp.sqrt(ss), EPS)
        u = lax.dot_general(tn, tn, (((0,), (0,)), ((), ())))
        row = lax.broadcasted_iota(jnp.int32, (A, A), 0)
        col = lax.broadcasted_iota(jnp.int32, (A, A), 1)
        eye = jnp.where(row == col, 1.0, 0.0)
        ul = jnp.sum((u - eye) ** 2, keepdims=True) * (1.0 / (A * A))
        jl = jin_ref[...] * (1.0 / (N * NEG))
        ab = ul + jl
        rl = rl_ref[...] * (1.0 / P)
        rl_ref[...] = rl
        ab_ref[...] = ab
        obj_ref[...] = rl + ab


def _main_call(ew2, rp, zn2, mw, ww, wb2, tw, t1, s1, t2, s):
    return pl.pallas_call(
        _main_body,
        grid=(N // BN,),
        in_specs=[
            pl.BlockSpec((BN, L * D), lambda i: (i, 0)),
            pl.BlockSpec((BN, D), lambda i: (i, 0)),
            pl.BlockSpec((BN, NEG * D), lambda i: (i, 0)),
            pl.BlockSpec((D, D), lambda i: (0, 0)),
            pl.BlockSpec((A, D), lambda i: (0, 0)),
            pl.BlockSpec((1, A), lambda i: (0, 0)),
            pl.BlockSpec((D, A), lambda i: (0, 0)),
            pl.BlockSpec((D, L * D), lambda i: (0, 0)),
            pl.BlockSpec((L * D, L), lambda i: (0, 0)),
            pl.BlockSpec((L, L * D), lambda i: (0, 0)),
            pl.BlockSpec((L * D, D), lambda i: (0, 0)),
        ],
        out_specs=[
            pl.BlockSpec((BN, 2 * D), lambda i: (i, 0)),
            pl.BlockSpec((1, 1), lambda i: (0, 0)),
        ],
        out_shape=[
            jax.ShapeDtypeStruct((N, 2 * D), jnp.float32),
            jax.ShapeDtypeStruct((1, 1), jnp.float32),
        ],
        compiler_params=pltpu.CompilerParams(
            dimension_semantics=("arbitrary",)),
    )(ew2, rp, zn2, mw, ww, wb2, tw, t1, s1, t2, s)


def _final_call(gh, lab, tw, jsum):
    nb = 4
    bp = P // nb
    return pl.pallas_call(
        _final_body,
        grid=(nb,),
        in_specs=[
            pl.BlockSpec((bp * H, 2 * D), lambda i: (i, 0)),
            pl.BlockSpec((bp * H, 2 * D), lambda i: (i + nb, 0)),
            pl.BlockSpec((bp, 1), lambda i: (i, 0)),
            pl.BlockSpec((D, A), lambda i: (0, 0)),
            pl.BlockSpec((1, 1), lambda i: (0, 0)),
        ],
        out_specs=[
            pl.BlockSpec((1, 1), lambda i: (0, 0)),
            pl.BlockSpec((1, 1), lambda i: (0, 0)),
            pl.BlockSpec((1, 1), lambda i: (0, 0)),
        ],
        out_shape=[
            jax.ShapeDtypeStruct((1, 1), jnp.float32),
            jax.ShapeDtypeStruct((1, 1), jnp.float32),
            jax.ShapeDtypeStruct((1, 1), jnp.float32),
        ],
        compiler_params=pltpu.CompilerParams(
            dimension_semantics=("arbitrary",)),
    )(gh, gh, lab, tw, jsum)


def _constants():
    jidx = jnp.arange(L * D)
    t1 = (jidx[None, :] % D == jnp.arange(D)[:, None]).astype(jnp.float32)
    s1 = (jidx[:, None] // D == jnp.arange(L)[None, :]).astype(jnp.float32)
    t2 = (jidx[None, :] % L == jnp.arange(L)[:, None]).astype(jnp.float32)
    s = (jidx[:, None] // L == jnp.arange(D)[None, :]).astype(jnp.float32)
    return t1, s1, t2, s


def kernel(historical_review, review_positive, review_negative, user, item,
           label, user_histor_idx, item_histor_idx,
           emb_table, M_w, W_w, W_b, T_w):
    hr = historical_review.reshape(-1).astype(jnp.int32)
    n_tiles = emb_table.shape[0] // 8
    emb3 = emb_table.reshape(n_tiles, 8, D)
    ew3 = _make_sc_rowdma_gather(N, n_tiles)(emb3, hr)
    ew2 = ew3.reshape(N, L * D)
    zn2 = review_negative.reshape(N, NEG * D)
    wb2 = W_b.reshape(1, A)
    t1, s1, t2, s = _constants()
    rs, jsum = _main_call(ew2, review_positive, zn2, M_w, W_w, wb2, T_w,
                          t1, s1, t2, s)
    hidx = jnp.concatenate([user_histor_idx.reshape(-1),
                            item_histor_idx.reshape(-1)]).astype(jnp.int32)
    gh = _make_sc_gather(2 * P * H)(rs, hidx)
    lab = label.reshape(P, 1)
    obj, rl, ab = _final_call(gh, lab, T_w, jsum)
    return (obj[0, 0], rl[0, 0], ab[0, 0])


# speedup vs baseline: 2.3109x; 1.0647x over previous
import functools

import jax
import jax.numpy as jnp
from jax import lax
from jax.experimental import pallas as pl
from jax.experimental.pallas import tpu as pltpu
from jax.experimental.pallas import tpu_sc as plsc

N = 4096
L = 50
D = 64
A = 32
NEG = 5
P = 1024
H = 8
AVG_RATING = 3.5
EPS = 1e-12
BN = 256
NC = 2
NS = 16
NW = NC * NS
GCH = 128


GRP = 8 * L


@functools.lru_cache(maxsize=None)
def _make_sc_rowdma_gather(n_rev: int, n_tiles: int, row_off: int = 0):
    rev_per_w = n_rev // NW
    nchunk = rev_per_w // 8
    mesh = plsc.VectorSubcoreMesh(core_axis_name="c", subcore_axis_name="s")

    @functools.partial(
        pl.kernel,
        mesh=mesh,
        out_type=jax.ShapeDtypeStruct((n_rev // 8, 8, L * D), jnp.float32),
        scratch_types=[
            pltpu.VMEM((GRP,), jnp.int32),
            pltpu.VMEM((8, L * D), jnp.float32),
            pltpu.SemaphoreType.DMA,
        ],
    )
    def gather_kernel(table_hbm, idx_hbm, out_hbm, idx_v, rows_v, sem):
        wid = lax.axis_index("s") * NC + lax.axis_index("c")
        base = row_off + wid * rev_per_w * L
        gbase = wid * nchunk

        def chunk_body(c, carry):
            off = base + c * GRP
            pltpu.sync_copy(idx_hbm.at[pl.ds(off, GRP)], idx_v)

            def fire16(c2, qr):
                q, r = qr
                vec = idx_v[pl.ds(c2 * 16, 16)]
                for j in range(16):
                    i = vec[j]
                    pltpu.make_async_copy(
                        table_hbm.at[i // 8, i % 8],
                        rows_v.at[q, pl.ds(r * D, D)], sem).start()
                    wrap = r == (L - 1)
                    q = jnp.where(wrap, q + 1, q)
                    r = jnp.where(wrap, 0, r + 1)
                return (q, r)

            lax.fori_loop(0, GRP // 16, fire16,
                          (jnp.int32(0), jnp.int32(0)))
            pltpu.make_async_copy(
                out_hbm.at[gbase + c], rows_v, sem).wait()
            for s in range(8):
                pltpu.sync_copy(rows_v.at[s], out_hbm.at[gbase + c, s])
            return carry

        lax.fori_loop(0, nchunk, chunk_body, 0)

    return gather_kernel


@functools.lru_cache(maxsize=None)
def _make_sc_gather(n_idx: int):
    b_per_w = n_idx // NW
    nchunk = b_per_w // GCH
    mesh = plsc.VectorSubcoreMesh(core_axis_name="c", subcore_axis_name="s")

    @functools.partial(
        pl.kernel,
        mesh=mesh,
        out_type=jax.ShapeDtypeStruct((n_idx, 2 * D), jnp.float32),
        scratch_types=[
            pltpu.VMEM((GCH,), jnp.int32),
            pltpu.VMEM((GCH, 2 * D), jnp.float32),
            pltpu.SemaphoreType.DMA,
        ],
    )
    def gather_kernel(table_hbm, idx_hbm, out_hbm, idx_v, rows_v, sem):
        wid = lax.axis_index("s") * NC + lax.axis_index("c")
        base = wid * b_per_w

        def body(c, carry):
            off = base + c * GCH
            pltpu.sync_copy(idx_hbm.at[pl.ds(off, GCH)], idx_v)
            pltpu.async_copy(table_hbm.at[idx_v], rows_v, sem).wait()
            pltpu.sync_copy(rows_v, out_hbm.at[pl.ds(off, GCH)])
            return carry

        lax.fori_loop(0, nchunk, body, 0)

    return gather_kernel


def _main_body(ew_ref, rp_ref, zn_ref, mw_ref, ww_ref, wb_ref, tw_ref,
               t1_ref, s1_ref, t2_ref, s_ref, rs_ref, j_ref):
    ew = ew_ref[...]
    q = jnp.dot(rp_ref[...], mw_ref[...])
    qt = jnp.dot(q, t1_ref[...])
    dx = jnp.dot(ew * qt, s1_ref[...])
    m = jnp.max(dx, axis=1, keepdims=True)
    e = jnp.exp(dx - m)
    ax = e / jnp.sum(e, axis=1, keepdims=True)
    axt = jnp.dot(ax, t2_ref[...])
    zs = jnp.dot(ew * axt, s_ref[...])
    logits = lax.dot_general(zs, ww_ref[...],
                             (((1,), (1,)), ((), ()))) + wb_ref[...]
    pm = jnp.max(logits, axis=1, keepdims=True)
    pe = jnp.exp(logits - pm)
    pt = pe / jnp.sum(pe, axis=1, keepdims=True)
    rs = lax.dot_general(pt, tw_ref[...], (((1,), (1,)), ((), ())))
    rs_ref[:, :D] = rs
    rs_ref[:, D:] = jnp.zeros((BN, D), jnp.float32)
    rnorm = jnp.maximum(jnp.sqrt(jnp.sum(rs * rs, axis=1, keepdims=True)), EPS)
    znorm = jnp.maximum(jnp.sqrt(jnp.sum(zs * zs, axis=1, keepdims=True)), EPS)
    c1 = jnp.sum(rs * zs, axis=1, keepdims=True) / (rnorm * znorm)
    rhat = rs / rnorm
    acc = jnp.zeros((BN, 1), jnp.float32)
    for g in range(NEG):
        seg = zn_ref[:, g * D:(g + 1) * D]
        snorm = jnp.maximum(
            jnp.sqrt(jnp.sum(seg * seg, axis=1, keepdims=True)), EPS)
        c2 = jnp.sum(seg * rhat, axis=1, keepdims=True) / snorm
        acc = acc + jnp.maximum(c2 - c1, 0.0)

    @pl.when(pl.program_id(0) == 0)
    def _init():
        j_ref[...] = jnp.zeros_like(j_ref)

    j_ref[...] += jnp.sum(acc, keepdims=True)


def _final_body(gu_ref, gi_ref, lab_ref, tw_ref, jin_ref,
                obj_ref, rl_ref, ab_ref):
    i = pl.program_id(0)
    nb = pl.num_programs(0)
    bp = P // 4
    ue = jnp.sum(gu_ref[...].reshape(bp, H, 2 * D), axis=1) * (1.0 / H)
    ie = jnp.sum(gi_ref[...].reshape(bp, H, 2 * D), axis=1) * (1.0 / H)
    pred = jnp.sum(ue * ie, axis=1, keepdims=True) + AVG_RATING
    diff = pred - lab_ref[...]

    @pl.when(i == 0)
    def _init():
        rl_ref[...] = jnp.zeros_like(rl_ref)

    rl_ref[...] += jnp.sum(diff * diff, keepdims=True)

    @pl.when(i == nb - 1)
    def _fin():
        tw = tw_ref[...]
        ss = jnp.sum(tw * tw, axis=0, keepdims=True)
        tn = tw / jnp.maximum(jnp.sqrt(ss), EPS)
        u = lax.dot_general(tn, tn, (((0,), (0,)), ((), ())))
        row = lax.broadcasted_iota(jnp.int32, (A, A), 0)
        col = lax.broadcasted_iota(jnp.int32, (A, A), 1)
        eye = jnp.where(row == col, 1.0, 0.0)
        ul = jnp.sum((u - eye) ** 2, keepdims=True) * (1.0 / (A * A))
        jl = jin_ref[...] * (1.0 / (N * NEG))
        ab = ul + jl
        rl = rl_ref[...] * (1.0 / P)
        rl_ref[...] = rl
        ab_ref[...] = ab
        obj_ref[...] = rl + ab


def _main_call(ew2, rp, zn2, mw, ww, wb2, tw, t1, s1, t2, s, blk0, nblk):
    return pl.pallas_call(
        _main_body,
        grid=(nblk,),
        in_specs=[
            pl.BlockSpec((BN, L * D), lambda i: (i, 0)),
            pl.BlockSpec((BN, D), lambda i: (i + blk0, 0)),
            pl.BlockSpec((BN, NEG * D), lambda i: (i + blk0, 0)),
            pl.BlockSpec((D, D), lambda i: (0, 0)),
            pl.BlockSpec((A, D), lambda i: (0, 0)),
            pl.BlockSpec((1, A), lambda i: (0, 0)),
            pl.BlockSpec((D, A), lambda i: (0, 0)),
            pl.BlockSpec((D, L * D), lambda i: (0, 0)),
            pl.BlockSpec((L * D, L), lambda i: (0, 0)),
            pl.BlockSpec((L, L * D), lambda i: (0, 0)),
            pl.BlockSpec((L * D, D), lambda i: (0, 0)),
        ],
        out_specs=[
            pl.BlockSpec((BN, 2 * D), lambda i: (i, 0)),
            pl.BlockSpec((1, 1), lambda i: (0, 0)),
        ],
        out_shape=[
            jax.ShapeDtypeStruct((nblk * BN, 2 * D), jnp.float32),
            jax.ShapeDtypeStruct((1, 1), jnp.float32),
        ],
        compiler_params=pltpu.CompilerParams(
            dimension_semantics=("arbitrary",)),
    )(ew2, rp, zn2, mw, ww, wb2, tw, t1, s1, t2, s)


def _final_call(gh, lab, tw, jsum):
    nb = 4
    bp = P // nb
    return pl.pallas_call(
        _final_body,
        grid=(nb,),
        in_specs=[
            pl.BlockSpec((bp * H, 2 * D), lambda i: (i, 0)),
            pl.BlockSpec((bp * H, 2 * D), lambda i: (i + nb, 0)),
            pl.BlockSpec((bp, 1), lambda i: (i, 0)),
            pl.BlockSpec((D, A), lambda i: (0, 0)),
            pl.BlockSpec((1, 1), lambda i: (0, 0)),
        ],
        out_specs=[
            pl.BlockSpec((1, 1), lambda i: (0, 0)),
            pl.BlockSpec((1, 1), lambda i: (0, 0)),
            pl.BlockSpec((1, 1), lambda i: (0, 0)),
        ],
        out_shape=[
            jax.ShapeDtypeStruct((1, 1), jnp.float32),
            jax.ShapeDtypeStruct((1, 1), jnp.float32),
            jax.ShapeDtypeStruct((1, 1), jnp.float32),
        ],
        compiler_params=pltpu.CompilerParams(
            dimension_semantics=("arbitrary",)),
    )(gh, gh, lab, tw, jsum)


def _constants():
    jidx = jnp.arange(L * D)
    t1 = (jidx[None, :] % D == jnp.arange(D)[:, None]).astype(jnp.float32)
    s1 = (jidx[:, None] // D == jnp.arange(L)[None, :]).astype(jnp.float32)
    t2 = (jidx[None, :] % L == jnp.arange(L)[:, None]).astype(jnp.float32)
    s = (jidx[:, None] // L == jnp.arange(D)[None, :]).astype(jnp.float32)
    return t1, s1, t2, s


def kernel(historical_review, review_positive, review_negative, user, item,
           label, user_histor_idx, item_histor_idx,
           emb_table, M_w, W_w, W_b, T_w):
    hr = historical_review.reshape(-1).astype(jnp.int32)
    n_tiles = emb_table.shape[0] // 8
    emb3 = emb_table.reshape(n_tiles, 8, D)
    zn2 = review_negative.reshape(N, NEG * D)
    wb2 = W_b.reshape(1, A)
    t1, s1, t2, s = _constants()
    nsplit = 4
    nsub = N // nsplit
    rs_parts, j_parts = [], []
    for k in range(nsplit):
        g = _make_sc_rowdma_gather(nsub, n_tiles, k * nsub * L)
        ew3 = g(emb3, hr)
        ew2 = ew3.reshape(nsub, L * D)
        rs_k, j_k = _main_call(ew2, review_positive, zn2, M_w, W_w, wb2,
                               T_w, t1, s1, t2, s,
                               k * (nsub // BN), nsub // BN)
        rs_parts.append(rs_k)
        j_parts.append(j_k)
    rs = jnp.concatenate(rs_parts, axis=0)
    jsum = j_parts[0] + j_parts[1] + j_parts[2] + j_parts[3]
    hidx = jnp.concatenate([user_histor_idx.reshape(-1),
                            item_histor_idx.reshape(-1)]).astype(jnp.int32)
    gh = _make_sc_gather(2 * P * H)(rs, hidx)
    lab = label.reshape(P, 1)
    obj, rl, ab = _final_call(gh, lab, T_w, jsum)
    return (obj[0, 0], rl[0, 0], ab[0, 0])


# speedup vs baseline: 2.3152x; 1.0018x over previous
import functools

import jax
import jax.numpy as jnp
from jax import lax
from jax.experimental import pallas as pl
from jax.experimental.pallas import tpu as pltpu
from jax.experimental.pallas import tpu_sc as plsc

N = 4096
L = 50
D = 64
A = 32
NEG = 5
P = 1024
H = 8
AVG_RATING = 3.5
EPS = 1e-12
BN = 256
NC = 2
NS = 16
NW = NC * NS
GCH = 128


GRP = 8 * L


@functools.lru_cache(maxsize=None)
def _make_sc_rowdma_gather(n_rev: int, n_tiles: int, row_off: int = 0):
    rev_per_w = n_rev // NW
    nchunk = rev_per_w // 8
    mesh = plsc.VectorSubcoreMesh(core_axis_name="c", subcore_axis_name="s")

    @functools.partial(
        pl.kernel,
        mesh=mesh,
        out_type=jax.ShapeDtypeStruct((n_rev // 8, 8, L * D), jnp.float32),
        scratch_types=[
            pltpu.VMEM((GRP,), jnp.int32),
            pltpu.VMEM((1, GRP * D), jnp.float32),
            pltpu.SemaphoreType.DMA,
        ],
    )
    def gather_kernel(table_hbm, idx_hbm, out_hbm, idx_v, rows_v, sem):
        wid = lax.axis_index("s") * NC + lax.axis_index("c")
        base = row_off + wid * rev_per_w * L
        gbase = wid * nchunk

        def chunk_body(c, carry):
            off = base + c * GRP
            pltpu.sync_copy(idx_hbm.at[pl.ds(off, GRP)], idx_v)

            def fire16(c2, carry2):
                vec = idx_v[pl.ds(c2 * 16, 16)]
                kb = c2 * (16 * D)
                for j in range(16):
                    i = vec[j]
                    pltpu.make_async_copy(
                        table_hbm.at[i // 8, i % 8],
                        rows_v.at[0, pl.ds(kb + j * D, D)], sem).start()
                return carry2

            lax.fori_loop(0, GRP // 16, fire16, 0)
            for s in range(8):
                pltpu.make_async_copy(
                    out_hbm.at[gbase + c, s],
                    rows_v.at[0, pl.ds(s * L * D, L * D)], sem).wait()
            for s in range(8):
                pltpu.sync_copy(rows_v.at[0, pl.ds(s * L * D, L * D)],
                                out_hbm.at[gbase + c, s])
            return carry

        lax.fori_loop(0, nchunk, chunk_body, 0)

    return gather_kernel


@functools.lru_cache(maxsize=None)
def _make_sc_gather(n_idx: int):
    b_per_w = n_idx // NW
    nchunk = b_per_w // GCH
    mesh = plsc.VectorSubcoreMesh(core_axis_name="c", subcore_axis_name="s")

    @functools.partial(
        pl.kernel,
        mesh=mesh,
        out_type=jax.ShapeDtypeStruct((n_idx, 2 * D), jnp.float32),
        scratch_types=[
            pltpu.VMEM((GCH,), jnp.int32),
            pltpu.VMEM((GCH, 2 * D), jnp.float32),
            pltpu.SemaphoreType.DMA,
        ],
    )
    def gather_kernel(table_hbm, idx_hbm, out_hbm, idx_v, rows_v, sem):
        wid = lax.axis_index("s") * NC + lax.axis_index("c")
        base = wid * b_per_w

        def body(c, carry):
            off = base + c * GCH
            pltpu.sync_copy(idx_hbm.at[pl.ds(off, GCH)], idx_v)
            pltpu.async_copy(table_hbm.at[idx_v], rows_v, sem).wait()
            pltpu.sync_copy(rows_v, out_hbm.at[pl.ds(off, GCH)])
            return carry

        lax.fori_loop(0, nchunk, body, 0)

    return gather_kernel


def _main_body(ew_ref, rp_ref, zn_ref, mw_ref, ww_ref, wb_ref, tw_ref,
               t1_ref, s1_ref, t2_ref, s_ref, rs_ref, j_ref):
    ew = ew_ref[...]
    q = jnp.dot(rp_ref[...], mw_ref[...])
    qt = jnp.dot(q, t1_ref[...])
    dx = jnp.dot(ew * qt, s1_ref[...])
    m = jnp.max(dx, axis=1, keepdims=True)
    e = jnp.exp(dx - m)
    ax = e / jnp.sum(e, axis=1, keepdims=True)
    axt = jnp.dot(ax, t2_ref[...])
    zs = jnp.dot(ew * axt, s_ref[...])
    logits = lax.dot_general(zs, ww_ref[...],
                             (((1,), (1,)), ((), ()))) + wb_ref[...]
    pm = jnp.max(logits, axis=1, keepdims=True)
    pe = jnp.exp(logits - pm)
    pt = pe / jnp.sum(pe, axis=1, keepdims=True)
    rs = lax.dot_general(pt, tw_ref[...], (((1,), (1,)), ((), ())))
    rs_ref[:, :D] = rs
    rs_ref[:, D:] = jnp.zeros((BN, D), jnp.float32)
    rnorm = jnp.maximum(jnp.sqrt(jnp.sum(rs * rs, axis=1, keepdims=True)), EPS)
    znorm = jnp.maximum(jnp.sqrt(jnp.sum(zs * zs, axis=1, keepdims=True)), EPS)
    c1 = jnp.sum(rs * zs, axis=1, keepdims=True) / (rnorm * znorm)
    rhat = rs / rnorm
    acc = jnp.zeros((BN, 1), jnp.float32)
    for g in range(NEG):
        seg = zn_ref[:, g * D:(g + 1) * D]
        snorm = jnp.maximum(
            jnp.sqrt(jnp.sum(seg * seg, axis=1, keepdims=True)), EPS)
        c2 = jnp.sum(seg * rhat, axis=1, keepdims=True) / snorm
        acc = acc + jnp.maximum(c2 - c1, 0.0)

    @pl.when(pl.program_id(0) == 0)
    def _init():
        j_ref[...] = jnp.zeros_like(j_ref)

    j_ref[...] += jnp.sum(acc, keepdims=True)


def _final_body(gu_ref, gi_ref, lab_ref, tw_ref, jin_ref,
                obj_ref, rl_ref, ab_ref):
    i = pl.program_id(0)
    nb = pl.num_programs(0)
    bp = P // 4
    ue = jnp.sum(gu_ref[...].reshape(bp, H, 2 * D), axis=1) * (1.0 / H)
    ie = jnp.sum(gi_ref[...].reshape(bp, H, 2 * D), axis=1) * (1.0 / H)
    pred = jnp.sum(ue * ie, axis=1, keepdims=True) + AVG_RATING
    diff = pred - lab_ref[...]

    @pl.when(i == 0)
    def _init():
        rl_ref[...] = jnp.zeros_like(rl_ref)

    rl_ref[...] += jnp.sum(diff * diff, keepdims=True)

    @pl.when(i == nb - 1)
    def _fin():
        tw = tw_ref[...]
        ss = jnp.sum(tw * tw, axis=0, keepdims=True)
        tn = tw / jnp.maximum(jnp.sqrt(ss), EPS)
        u = lax.dot_general(tn, tn, (((0,), (0,)), ((), ())))
        row = lax.broadcasted_iota(jnp.int32, (A, A), 0)
        col = lax.broadcasted_iota(jnp.int32, (A, A), 1)
        eye = jnp.where(row == col, 1.0, 0.0)
        ul = jnp.sum((u - eye) ** 2, keepdims=True) * (1.0 / (A * A))
        jl = jin_ref[...] * (1.0 / (N * NEG))
        ab = ul + jl
        rl = rl_ref[...] * (1.0 / P)
        rl_ref[...] = rl
        ab_ref[...] = ab
        obj_ref[...] = rl + ab


def _main_call(ew2, rp, zn2, mw, ww, wb2, tw, t1, s1, t2, s, blk0, nblk):
    return pl.pallas_call(
        _main_body,
        grid=(nblk,),
        in_specs=[
            pl.BlockSpec((BN, L * D), lambda i: (i, 0)),
            pl.BlockSpec((BN, D), lambda i: (i + blk0, 0)),
            pl.BlockSpec((BN, NEG * D), lambda i: (i + blk0, 0)),
            pl.BlockSpec((D, D), lambda i: (0, 0)),
            pl.BlockSpec((A, D), lambda i: (0, 0)),
            pl.BlockSpec((1, A), lambda i: (0, 0)),
            pl.BlockSpec((D, A), lambda i: (0, 0)),
            pl.BlockSpec((D, L * D), lambda i: (0, 0)),
            pl.BlockSpec((L * D, L), lambda i: (0, 0)),
            pl.BlockSpec((L, L * D), lambda i: (0, 0)),
            pl.BlockSpec((L * D, D), lambda i: (0, 0)),
        ],
        out_specs=[
            pl.BlockSpec((BN, 2 * D), lambda i: (i, 0)),
            pl.BlockSpec((1, 1), lambda i: (0, 0)),
        ],
        out_shape=[
            jax.ShapeDtypeStruct((nblk * BN, 2 * D), jnp.float32),
            jax.ShapeDtypeStruct((1, 1), jnp.float32),
        ],
        compiler_params=pltpu.CompilerParams(
            dimension_semantics=("arbitrary",)),
    )(ew2, rp, zn2, mw, ww, wb2, tw, t1, s1, t2, s)


def _final_call(gh, lab, tw, jsum):
    nb = 4
    bp = P // nb
    return pl.pallas_call(
        _final_body,
        grid=(nb,),
        in_specs=[
            pl.BlockSpec((bp * H, 2 * D), lambda i: (i, 0)),
            pl.BlockSpec((bp * H, 2 * D), lambda i: (i + nb, 0)),
            pl.BlockSpec((bp, 1), lambda i: (i, 0)),
            pl.BlockSpec((D, A), lambda i: (0, 0)),
            pl.BlockSpec((1, 1), lambda i: (0, 0)),
        ],
        out_specs=[
            pl.BlockSpec((1, 1), lambda i: (0, 0)),
            pl.BlockSpec((1, 1), lambda i: (0, 0)),
            pl.BlockSpec((1, 1), lambda i: (0, 0)),
        ],
        out_shape=[
            jax.ShapeDtypeStruct((1, 1), jnp.float32),
            jax.ShapeDtypeStruct((1, 1), jnp.float32),
            jax.ShapeDtypeStruct((1, 1), jnp.float32),
        ],
        compiler_params=pltpu.CompilerParams(
            dimension_semantics=("arbitrary",)),
    )(gh, gh, lab, tw, jsum)


def _constants():
    jidx = jnp.arange(L * D)
    t1 = (jidx[None, :] % D == jnp.arange(D)[:, None]).astype(jnp.float32)
    s1 = (jidx[:, None] // D == jnp.arange(L)[None, :]).astype(jnp.float32)
    t2 = (jidx[None, :] % L == jnp.arange(L)[:, None]).astype(jnp.float32)
    s = (jidx[:, None] // L == jnp.arange(D)[None, :]).astype(jnp.float32)
    return t1, s1, t2, s


def kernel(historical_review, review_positive, review_negative, user, item,
           label, user_histor_idx, item_histor_idx,
           emb_table, M_w, W_w, W_b, T_w):
    hr = historical_review.reshape(-1).astype(jnp.int32)
    n_tiles = emb_table.shape[0] // 8
    emb3 = emb_table.reshape(n_tiles, 8, D)
    zn2 = review_negative.reshape(N, NEG * D)
    wb2 = W_b.reshape(1, A)
    t1, s1, t2, s = _constants()
    nsplit = 4
    nsub = N // nsplit
    rs_parts, j_parts = [], []
    for k in range(nsplit):
        g = _make_sc_rowdma_gather(nsub, n_tiles, k * nsub * L)
        ew3 = g(emb3, hr)
        ew2 = ew3.reshape(nsub, L * D)
        rs_k, j_k = _main_call(ew2, review_positive, zn2, M_w, W_w, wb2,
                               T_w, t1, s1, t2, s,
                               k * (nsub // BN), nsub // BN)
        rs_parts.append(rs_k)
        j_parts.append(j_k)
    rs = jnp.concatenate(rs_parts, axis=0)
    jsum = j_parts[0] + j_parts[1] + j_parts[2] + j_parts[3]
    hidx = jnp.concatenate([user_histor_idx.reshape(-1),
                            item_histor_idx.reshape(-1)]).astype(jnp.int32)
    gh = _make_sc_gather(2 * P * H)(rs, hidx)
    lab = label.reshape(P, 1)
    obj, rl, ab = _final_call(gh, lab, T_w, jsum)
    return (obj[0, 0], rl[0, 0], ab[0, 0])
